# Initial kernel scaffold; baseline (speedup 1.0000x reference)
import jax
import jax.numpy as jnp
from jax.experimental import pallas as pl


def kernel(tokens, table):
    raise NotImplementedError("write your pallas kernel here")


# speedup vs baseline: 1.4830x; 1.4830x over previous
import functools

import jax
import jax.numpy as jnp
from jax import lax
from jax.experimental import pallas as pl
from jax.experimental.pallas import tpu as pltpu
from jax.experimental.pallas import tpu_sc as plsc

_B = 4096 * 200
_D = 32
_NC, _NS = 2, 16
_NW = _NC * _NS
_BPW = _B // _NW
_C = 1280
_NCHUNK = _BPW // _C


def _build():
    mesh = plsc.VectorSubcoreMesh(core_axis_name="c", subcore_axis_name="s")

    @functools.partial(
        pl.kernel,
        mesh=mesh,
        out_type=jax.ShapeDtypeStruct((_B, _D), jnp.float32),
        compiler_params=pltpu.CompilerParams(use_tc_tiling_on_sc=False),
        scratch_types=[
            pltpu.VMEM((_BPW,), jnp.int32),
            pltpu.VMEM((_C, _D), jnp.float32),
            pltpu.SemaphoreType.DMA,
        ],
    )
    def gather_kernel(idx_hbm, table_hbm, out_hbm, idx_v, rows_v, gsem):
        wid = lax.axis_index("s") * _NC + lax.axis_index("c")
        base = wid * _BPW
        pltpu.sync_copy(idx_hbm.at[pl.ds(base, _BPW)], idx_v)

        def body(c, carry):
            off = c * _C
            pltpu.async_copy(
                table_hbm.at[idx_v.at[pl.ds(off, _C)]], rows_v, gsem
            ).wait()
            pltpu.sync_copy(rows_v, out_hbm.at[pl.ds(base + off, _C)])
            return carry

        lax.fori_loop(0, _NCHUNK, body, 0)

    return gather_kernel


_GATHER = _build()


def kernel(tokens, table):
    idx = tokens.reshape(-1).astype(jnp.int32)
    out = _GATHER(idx, table)
    return out.reshape(tokens.shape + (table.shape[1],))


# speedup vs baseline: 1.4999x; 1.0114x over previous
import functools

import jax
import jax.numpy as jnp
from jax import lax
from jax.experimental import pallas as pl
from jax.experimental.pallas import tpu as pltpu
from jax.experimental.pallas import tpu_sc as plsc

_B = 4096 * 200
_D = 32
_NC, _NS = 2, 16
_NW = _NC * _NS
_BPW = _B // _NW
_C = 1280
_NCHUNK = _BPW // _C


def _build():
    mesh = plsc.VectorSubcoreMesh(core_axis_name="c", subcore_axis_name="s")

    @functools.partial(
        pl.kernel,
        mesh=mesh,
        out_type=jax.ShapeDtypeStruct((_B, _D), jnp.float32),
        compiler_params=pltpu.CompilerParams(use_tc_tiling_on_sc=False),
        scratch_types=[
            pltpu.VMEM((_BPW,), jnp.int32),
            pltpu.VMEM((_C, _D), jnp.float32),
            pltpu.VMEM((_C, _D), jnp.float32),
            pltpu.SemaphoreType.DMA,
            pltpu.SemaphoreType.DMA,
            pltpu.SemaphoreType.DMA,
            pltpu.SemaphoreType.DMA,
        ],
    )
    def gather_kernel(idx_hbm, table_hbm, out_hbm, idx_v, buf_a, buf_b,
                      gsem_a, gsem_b, wsem_a, wsem_b):
        wid = lax.axis_index("s") * _NC + lax.axis_index("c")
        base = wid * _BPW
        pltpu.sync_copy(idx_hbm.at[pl.ds(base, _BPW)], idx_v)

        bufs = (buf_a, buf_b)
        gsems = (gsem_a, gsem_b)
        wsems = (wsem_a, wsem_b)

        def start_gather(c, b):
            return pltpu.async_copy(
                table_hbm.at[idx_v.at[pl.ds(c * _C, _C)]], bufs[b], gsems[b])

        def start_write(c, b):
            return pltpu.async_copy(
                bufs[b], out_hbm.at[pl.ds(base + c * _C, _C)], wsems[b])

        start_gather(0, 0)
        for c in range(_NCHUNK):
            b = c & 1
            if c + 1 < _NCHUNK:
                if c >= 1:
                    pltpu.make_async_copy(
                        bufs[1 - b],
                        out_hbm.at[pl.ds(base + (c - 1) * _C, _C)],
                        wsems[1 - b]).wait()
                start_gather(c + 1, 1 - b)
            pltpu.make_async_copy(
                table_hbm.at[idx_v.at[pl.ds(c * _C, _C)]], bufs[b],
                gsems[b]).wait()
            start_write(c, b)
        for c in (_NCHUNK - 2, _NCHUNK - 1):
            pltpu.make_async_copy(
                bufs[c & 1], out_hbm.at[pl.ds(base + c * _C, _C)],
                wsems[c & 1]).wait()

    return gather_kernel


_GATHER = _build()


def kernel(tokens, table):
    idx = tokens.reshape(-1).astype(jnp.int32)
    out = _GATHER(idx, table)
    return out.reshape(tokens.shape + (table.shape[1],))
